# baseline (speedup 1.0000x reference)
import jax
import jax.numpy as jnp
from jax.experimental import pallas as pl
from jax.experimental.pallas import tpu as pltpu


def _split_dma_body(x_ref, o1_ref, o2_ref, sem1, sem2):
    nt = o1_ref.shape[1]
    c1 = pltpu.make_async_copy(x_ref.at[:, :nt], o1_ref, sem1)
    c2 = pltpu.make_async_copy(x_ref.at[:, nt:], o2_ref, sem2)
    c1.start()
    c2.start()
    c1.wait()
    c2.wait()


def kernel(x):
    B, C, H, W = x.shape
    HW = H * W
    nt = HW // 2
    rows = B * C
    xr = x.reshape(rows, HW)
    o1, o2 = pl.pallas_call(
        _split_dma_body,
        in_specs=[pl.BlockSpec(memory_space=pl.ANY)],
        out_specs=[
            pl.BlockSpec(memory_space=pl.ANY),
            pl.BlockSpec(memory_space=pl.ANY),
        ],
        out_shape=[
            jax.ShapeDtypeStruct((rows, nt), x.dtype),
            jax.ShapeDtypeStruct((rows, nt), x.dtype),
        ],
        scratch_shapes=[pltpu.SemaphoreType.DMA, pltpu.SemaphoreType.DMA],
    )(xr)
    X1 = o1.reshape(B, C, H, nt // W)
    X2 = o2.reshape(B, C, H, nt // W)
    return (X1, X2)


# speedup vs baseline: 5.2073x; 5.2073x over previous
import jax
import jax.numpy as jnp
from jax.experimental import pallas as pl
from jax.experimental.pallas import tpu as pltpu

_BLK = 512


def _split_body(x_ref, o1_ref, o2_ref):
    nt = o1_ref.shape[1]
    o1_ref[...] = x_ref[:, :nt]
    o2_ref[...] = x_ref[:, nt:]


def kernel(x):
    B, C, H, W = x.shape
    HW = H * W
    nt = HW // 2
    rows = B * C
    xr = x.reshape(rows, HW)
    grid = rows // _BLK
    o1, o2 = pl.pallas_call(
        _split_body,
        grid=(grid,),
        in_specs=[pl.BlockSpec((_BLK, HW), lambda i: (i, 0))],
        out_specs=[
            pl.BlockSpec((_BLK, nt), lambda i: (i, 0)),
            pl.BlockSpec((_BLK, nt), lambda i: (i, 0)),
        ],
        out_shape=[
            jax.ShapeDtypeStruct((rows, nt), x.dtype),
            jax.ShapeDtypeStruct((rows, nt), x.dtype),
        ],
    )(xr)
    X1 = o1.reshape(B, C, H, nt // W)
    X2 = o2.reshape(B, C, H, nt // W)
    return (X1, X2)
